# baseline (device time: 55108 ns/iter reference)
import jax
import jax.numpy as jnp
from jax import lax
from jax.experimental import pallas as pl
from jax.experimental.pallas import tpu as pltpu

N_DEV = 8


def kernel(x, router_W, route_idx, expert_W):
    n_tok, d_model = x.shape
    n_local, _, d_hidden = expert_W.shape
    rows_per = n_tok // N_DEV
    n_experts = router_W.shape[1]
    n_steps = N_DEV - 1

    half = rows_per // 2

    def body(x_ref, rw_ref, idx_ref, ew_ref, out_ref,
             wt_ref, ew_bf_ref, send_cw_ref, send_ccw_ref,
             recv_cw_ref, recv_ccw_ref,
             send_cw_sems, recv_cw_sems, send_ccw_sems, recv_ccw_sems):
        my = lax.axis_index("i")
        left = lax.rem(my + N_DEV - 1, N_DEV)
        right = lax.rem(my + 1, N_DEV)

        xf = x_ref[:, :]
        scores = jnp.dot(xf, rw_ref[:, :], preferred_element_type=jnp.float32)
        m = jnp.max(scores, axis=1, keepdims=True)
        p = jnp.exp(scores - m)
        denom = jnp.sum(p, axis=1, keepdims=True)
        probs = p / denom

        cols = lax.broadcasted_iota(jnp.int32, (n_tok, n_experts), 1)
        idx0 = idx_ref[:, 0:1]
        idx1 = idx_ref[:, 1:2]
        g0 = jnp.sum(jnp.where(cols == idx0, probs, 0.0), axis=1, keepdims=True)
        g1 = jnp.sum(jnp.where(cols == idx1, probs, 0.0), axis=1, keepdims=True)
        gs = g0 + g1
        w0 = g0 / gs
        w1 = g1 / gs
        for e in range(n_local):
            ge = my * n_local + e
            wt_ref[:, e:e + 1] = (jnp.where(idx0 == ge, w0, 0.0)
                                  + jnp.where(idx1 == ge, w1, 0.0))
        for e in range(n_local):
            ew_bf_ref[pl.ds(e * d_model, d_model), :] = ew_ref[e].astype(
                jnp.bfloat16)

        def compute_half(c, top):
            row0 = c * rows_per + (0 if top else half)
            xc = x_ref[pl.ds(row0, half), :]
            xs = jnp.concatenate(
                [xc * wt_ref[pl.ds(row0, half), e:e + 1] for e in range(n_local)],
                axis=1,
            ).astype(jnp.bfloat16)
            return jnp.dot(xs, ew_bf_ref[:, :],
                           preferred_element_type=jnp.float32)

        send_cw_ref[:, :] = compute_half(
            lax.rem(my + n_steps, N_DEV), True).astype(jnp.bfloat16)
        send_ccw_ref[:, :] = compute_half(
            lax.rem(my + 1, N_DEV), False).astype(jnp.bfloat16)

        barrier_sem = pltpu.get_barrier_semaphore()
        for nbr in (left, right):
            pl.semaphore_signal(
                barrier_sem, inc=1,
                device_id=(nbr,), device_id_type=pl.DeviceIdType.MESH,
            )
        pl.semaphore_wait(barrier_sem, 2)

        for s in range(n_steps):
            cw_recv_idx = lax.rem(my + (n_steps - s - 1), N_DEV)
            ccw_recv_idx = lax.rem(my + s + 2, N_DEV)
            rdma_cw = pltpu.make_async_remote_copy(
                src_ref=send_cw_ref,
                dst_ref=recv_cw_ref.at[s],
                send_sem=send_cw_sems.at[s],
                recv_sem=recv_cw_sems.at[s],
                device_id=(right,),
                device_id_type=pl.DeviceIdType.MESH,
            )
            rdma_ccw = pltpu.make_async_remote_copy(
                src_ref=send_ccw_ref,
                dst_ref=recv_ccw_ref.at[s],
                send_sem=send_ccw_sems.at[s],
                recv_sem=recv_ccw_sems.at[s],
                device_id=(left,),
                device_id_type=pl.DeviceIdType.MESH,
            )
            rdma_cw.start()
            rdma_ccw.start()
            acc_cw = compute_half(cw_recv_idx, True)
            acc_ccw = compute_half(ccw_recv_idx, False)
            rdma_cw.wait()
            rdma_ccw.wait()
            comb_cw = recv_cw_ref[s].astype(jnp.float32) + acc_cw
            comb_ccw = recv_ccw_ref[s].astype(jnp.float32) + acc_ccw
            if s < n_steps - 1:
                send_cw_ref[:, :] = comb_cw.astype(jnp.bfloat16)
                send_ccw_ref[:, :] = comb_ccw.astype(jnp.bfloat16)
            else:
                out_ref[0:half, :] = comb_cw
                out_ref[half:rows_per, :] = comb_ccw

    return pl.pallas_call(
        body,
        out_shape=jax.ShapeDtypeStruct((rows_per, d_hidden), jnp.float32),
        in_specs=[
            pl.BlockSpec(memory_space=pltpu.VMEM),
            pl.BlockSpec(memory_space=pltpu.VMEM),
            pl.BlockSpec(memory_space=pltpu.VMEM),
            pl.BlockSpec(memory_space=pltpu.VMEM),
        ],
        out_specs=pl.BlockSpec(memory_space=pltpu.VMEM),
        scratch_shapes=[
            pltpu.VMEM((n_tok, n_local), jnp.float32),
            pltpu.VMEM((n_local * d_model, d_hidden), jnp.bfloat16),
            pltpu.VMEM((half, d_hidden), jnp.bfloat16),
            pltpu.VMEM((half, d_hidden), jnp.bfloat16),
            pltpu.VMEM((n_steps, half, d_hidden), jnp.bfloat16),
            pltpu.VMEM((n_steps, half, d_hidden), jnp.bfloat16),
            pltpu.SemaphoreType.DMA((n_steps,)),
            pltpu.SemaphoreType.DMA((n_steps,)),
            pltpu.SemaphoreType.DMA((n_steps,)),
            pltpu.SemaphoreType.DMA((n_steps,)),
        ],
        compiler_params=pltpu.CompilerParams(collective_id=0),
    )(x, router_W, route_idx, expert_W)


# device time: 47119 ns/iter; 1.1695x vs baseline; 1.1695x over previous
import jax
import jax.numpy as jnp
from jax import lax
from jax.experimental import pallas as pl
from jax.experimental.pallas import tpu as pltpu

N_DEV = 8


def kernel(x, router_W, route_idx, expert_W):
    n_tok, d_model = x.shape
    n_local, _, d_hidden = expert_W.shape
    rows_per = n_tok // N_DEV
    n_experts = router_W.shape[1]
    half = rows_per // 2

    A_MASKS, A_L = (1, 3, 4), ([0, 3, 4, 7], [0, 4], [0])
    B_MASKS, B_L = (4, 1, 3), ([0, 1, 3, 2], [0, 3], [0])

    def body(x_ref, rw_ref, idx_ref, ew_ref, out_ref,
             partial_ref, wt_ref, ew_bf_ref,
             sA1, rA1, sA2, rA2, sA3, rA3,
             sB1, rB1, sB2, rB2, sB3, rB3,
             sems_sA, sems_rA, sems_sB, sems_rB):
        my = lax.axis_index("i")

        xf = x_ref[:, :]
        scores = jnp.dot(xf, rw_ref[:, :], preferred_element_type=jnp.float32)
        m = jnp.max(scores, axis=1, keepdims=True)
        p = jnp.exp(scores - m)
        denom = jnp.sum(p, axis=1, keepdims=True)
        probs = p / denom

        cols = lax.broadcasted_iota(jnp.int32, (n_tok, n_experts), 1)
        idx0 = idx_ref[:, 0:1]
        idx1 = idx_ref[:, 1:2]
        g0 = jnp.sum(jnp.where(cols == idx0, probs, 0.0), axis=1, keepdims=True)
        g1 = jnp.sum(jnp.where(cols == idx1, probs, 0.0), axis=1, keepdims=True)
        gs = g0 + g1
        w0 = g0 / gs
        w1 = g1 / gs
        for e in range(n_local):
            ge = my * n_local + e
            wt_ref[:, e:e + 1] = (jnp.where(idx0 == ge, w0, 0.0)
                                  + jnp.where(idx1 == ge, w1, 0.0))
        for e in range(n_local):
            ew_bf_ref[pl.ds(e * d_model, d_model), :] = ew_ref[e].astype(
                jnp.bfloat16)

        def compute_half(c, top):
            row0 = c * rows_per + (0 if top else half)
            xc = x_ref[pl.ds(row0, half), :]
            xs = jnp.concatenate(
                [xc * wt_ref[pl.ds(row0, half), e:e + 1] for e in range(n_local)],
                axis=1,
            ).astype(jnp.bfloat16)
            return jnp.dot(xs, ew_bf_ref[:, :],
                           preferred_element_type=jnp.float32)

        def prow(c, top):
            return pl.ds(c * rows_per + (0 if top else half), half)

        def exchange(src, dst, ssem, rsem, partner):
            return pltpu.make_async_remote_copy(
                src_ref=src, dst_ref=dst, send_sem=ssem, recv_sem=rsem,
                device_id=(partner,), device_id_type=pl.DeviceIdType.MESH,
            )

        for k, l in enumerate(A_L[0]):
            sA1[k] = compute_half(my ^ (A_MASKS[0] ^ l), True).astype(jnp.bfloat16)
        for k, l in enumerate(B_L[0]):
            sB1[k] = compute_half(my ^ (B_MASKS[0] ^ l), False).astype(jnp.bfloat16)

        barrier_sem = pltpu.get_barrier_semaphore()
        for mask in (1, 3, 4):
            pl.semaphore_signal(
                barrier_sem, inc=1,
                device_id=(my ^ mask,), device_id_type=pl.DeviceIdType.MESH,
            )
        pl.semaphore_wait(barrier_sem, 3)

        xA1 = exchange(sA1, rA1, sems_sA.at[0], sems_rA.at[0], my ^ A_MASKS[0])
        xB1 = exchange(sB1, rB1, sems_sB.at[0], sems_rB.at[0], my ^ B_MASKS[0])
        xA1.start()
        xB1.start()

        for l in A_L[0]:
            partial_ref[prow(my ^ l, True), :] = compute_half(my ^ l, True)
        for l in B_L[0]:
            partial_ref[prow(my ^ l, False), :] = compute_half(my ^ l, False)

        xA1.wait()
        for k, l in enumerate(A_L[0]):
            r = prow(my ^ l, True)
            partial_ref[r, :] = partial_ref[r, :] + rA1[k].astype(jnp.float32)
        for k, l in enumerate(A_L[1]):
            sA2[k] = partial_ref[prow(my ^ A_MASKS[1] ^ l, True), :].astype(
                jnp.bfloat16)
        xA2 = exchange(sA2, rA2, sems_sA.at[1], sems_rA.at[1], my ^ A_MASKS[1])
        xA2.start()

        xB1.wait()
        for k, l in enumerate(B_L[0]):
            r = prow(my ^ l, False)
            partial_ref[r, :] = partial_ref[r, :] + rB1[k].astype(jnp.float32)
        for k, l in enumerate(B_L[1]):
            sB2[k] = partial_ref[prow(my ^ B_MASKS[1] ^ l, False), :].astype(
                jnp.bfloat16)
        xB2 = exchange(sB2, rB2, sems_sB.at[1], sems_rB.at[1], my ^ B_MASKS[1])
        xB2.start()

        xA2.wait()
        for k, l in enumerate(A_L[1]):
            r = prow(my ^ l, True)
            partial_ref[r, :] = partial_ref[r, :] + rA2[k].astype(jnp.float32)
        sA3[:, :] = partial_ref[prow(my ^ A_MASKS[2], True), :].astype(jnp.bfloat16)
        xA3 = exchange(sA3, rA3, sems_sA.at[2], sems_rA.at[2], my ^ A_MASKS[2])
        xA3.start()

        xB2.wait()
        for k, l in enumerate(B_L[1]):
            r = prow(my ^ l, False)
            partial_ref[r, :] = partial_ref[r, :] + rB2[k].astype(jnp.float32)
        sB3[:, :] = partial_ref[prow(my ^ B_MASKS[2], False), :].astype(jnp.bfloat16)
        xB3 = exchange(sB3, rB3, sems_sB.at[2], sems_rB.at[2], my ^ B_MASKS[2])
        xB3.start()

        xA3.wait()
        out_ref[0:half, :] = (partial_ref[prow(my, True), :]
                              + rA3[:, :].astype(jnp.float32))
        xB3.wait()
        out_ref[half:rows_per, :] = (partial_ref[prow(my, False), :]
                                     + rB3[:, :].astype(jnp.float32))

    return pl.pallas_call(
        body,
        out_shape=jax.ShapeDtypeStruct((rows_per, d_hidden), jnp.float32),
        in_specs=[
            pl.BlockSpec(memory_space=pltpu.VMEM),
            pl.BlockSpec(memory_space=pltpu.VMEM),
            pl.BlockSpec(memory_space=pltpu.VMEM),
            pl.BlockSpec(memory_space=pltpu.VMEM),
        ],
        out_specs=pl.BlockSpec(memory_space=pltpu.VMEM),
        scratch_shapes=[
            pltpu.VMEM((n_tok, d_hidden), jnp.float32),
            pltpu.VMEM((n_tok, n_local), jnp.float32),
            pltpu.VMEM((n_local * d_model, d_hidden), jnp.bfloat16),
            pltpu.VMEM((4, half, d_hidden), jnp.bfloat16),
            pltpu.VMEM((4, half, d_hidden), jnp.bfloat16),
            pltpu.VMEM((2, half, d_hidden), jnp.bfloat16),
            pltpu.VMEM((2, half, d_hidden), jnp.bfloat16),
            pltpu.VMEM((half, d_hidden), jnp.bfloat16),
            pltpu.VMEM((half, d_hidden), jnp.bfloat16),
            pltpu.VMEM((4, half, d_hidden), jnp.bfloat16),
            pltpu.VMEM((4, half, d_hidden), jnp.bfloat16),
            pltpu.VMEM((2, half, d_hidden), jnp.bfloat16),
            pltpu.VMEM((2, half, d_hidden), jnp.bfloat16),
            pltpu.VMEM((half, d_hidden), jnp.bfloat16),
            pltpu.VMEM((half, d_hidden), jnp.bfloat16),
            pltpu.SemaphoreType.DMA((3,)),
            pltpu.SemaphoreType.DMA((3,)),
            pltpu.SemaphoreType.DMA((3,)),
            pltpu.SemaphoreType.DMA((3,)),
        ],
        compiler_params=pltpu.CompilerParams(collective_id=0),
    )(x, router_W, route_idx, expert_W)
